# Initial kernel scaffold; baseline (speedup 1.0000x reference)
import jax
import jax.numpy as jnp
from jax.experimental import pallas as pl


def kernel(hours, days, months, holidays, hour_table, day_table, month_table, holiday_table, W, b):
    raise NotImplementedError("write your pallas kernel here")


# speedup vs baseline: 12.2490x; 12.2490x over previous
import functools

import jax
import jax.numpy as jnp
from jax import lax
from jax.experimental import pallas as pl
from jax.experimental.pallas import tpu as pltpu
from jax.experimental.pallas import tpu_sc as plsc

EMBED = 64
SUB = 16
NH, ND, NM, NO = 24, 7, 12, 2
NROWS = NH * ND * NM * NO
TOTAL = 4096 * 200
NC, NS = 2, 16
NW = NC * NS
PER_W = TOTAL // NW
CHUNK = 2560
BATCH = 128


def _fuse_body(ht_ref, dt_ref, mt_ref, ot_ref, w_ref, b_ref, out_ref):
    i = lax.broadcasted_iota(jnp.int32, (NROWS, 1), 0)
    parts = [
        (i // (ND * NM * NO), ht_ref, NH, 0),
        ((i // (NM * NO)) % ND, dt_ref, ND, 1),
        ((i // NO) % NM, mt_ref, NM, 2),
        (i % NO, ot_ref, NO, 3),
    ]
    acc = jnp.broadcast_to(b_ref[...], (NROWS, EMBED))
    for idx, tab_ref, n, j in parts:
        w_slice = w_ref[:, j * SUB:(j + 1) * SUB]
        pt = lax.dot_general(tab_ref[...], w_slice, (((1,), (1,)), ((), ())),
                             preferred_element_type=jnp.float32,
                             precision=lax.Precision.HIGHEST)
        oh = (idx == lax.broadcasted_iota(jnp.int32, (NROWS, n), 1)).astype(jnp.float32)
        acc = acc + jnp.dot(oh, pt, preferred_element_type=jnp.float32,
                            precision=lax.Precision.HIGHEST)
    out_ref[...] = acc


def _fuse_tables(ht, dt, mt, ot, W, b):
    return pl.pallas_call(
        _fuse_body,
        out_shape=jax.ShapeDtypeStruct((NROWS, EMBED), jnp.float32),
    )(ht, dt, mt, ot, W, b.reshape(1, EMBED))


@functools.cache
def _make_sc_gather():
    mesh = plsc.VectorSubcoreMesh(core_axis_name="c", subcore_axis_name="s")

    @functools.partial(
        pl.kernel,
        mesh=mesh,
        compiler_params=pltpu.CompilerParams(use_tc_tiling_on_sc=False),
        out_type=jax.ShapeDtypeStruct((TOTAL, EMBED), jnp.float32),
        scratch_types=[
            pltpu.VMEM((CHUNK,), jnp.int32),
            pltpu.VMEM((CHUNK,), jnp.int32),
            pltpu.VMEM((CHUNK,), jnp.int32),
            pltpu.VMEM((CHUNK,), jnp.int32),
            pltpu.VMEM((CHUNK,), jnp.int32),
            pltpu.VMEM((BATCH, EMBED), jnp.float32),
            pltpu.SemaphoreType.DMA,
        ],
    )
    def _sc_gather(h_hbm, d_hbm, m_hbm, o_hbm, tab_hbm, out_hbm,
                   h_v, d_v, m_v, o_v, f_v, rows_v, sem):
        wid = lax.axis_index("s") * NC + lax.axis_index("c")
        base = wid * PER_W

        def chunk_body(ci, _):
            start = base + ci * CHUNK
            pltpu.sync_copy(h_hbm.at[pl.ds(start, CHUNK)], h_v)
            pltpu.sync_copy(d_hbm.at[pl.ds(start, CHUNK)], d_v)
            pltpu.sync_copy(m_hbm.at[pl.ds(start, CHUNK)], m_v)
            pltpu.sync_copy(o_hbm.at[pl.ds(start, CHUNK)], o_v)

            def vec_body(vi, _):
                sl = pl.ds(vi * 16, 16)
                f_v[sl] = ((h_v[sl] * ND + d_v[sl]) * NM + m_v[sl]) * NO + o_v[sl]
                return 0

            lax.fori_loop(0, CHUNK // 16, vec_body, 0)

            def batch_body(bi, _):
                idx = f_v.at[pl.ds(bi * BATCH, BATCH)]
                pltpu.async_copy(tab_hbm.at[idx], rows_v, sem).wait()
                pltpu.sync_copy(rows_v, out_hbm.at[pl.ds(start + bi * BATCH, BATCH)])
                return 0

            lax.fori_loop(0, CHUNK // BATCH, batch_body, 0)
            return 0

        lax.fori_loop(0, PER_W // CHUNK, chunk_body, 0)

    return _sc_gather


def kernel(hours, days, months, holidays, hour_table, day_table, month_table,
           holiday_table, W, b):
    fused = _fuse_tables(hour_table, day_table, month_table, holiday_table, W, b)
    h = hours.reshape(TOTAL).astype(jnp.int32)
    d = days.reshape(TOTAL).astype(jnp.int32)
    m = months.reshape(TOTAL).astype(jnp.int32)
    o = holidays.reshape(TOTAL).astype(jnp.int32)
    out = _make_sc_gather()(h, d, m, o, fused)
    return out.reshape(hours.shape[0], hours.shape[1], EMBED)


# speedup vs baseline: 14.6595x; 1.1968x over previous
import functools

import jax
import jax.numpy as jnp
from jax import lax
from jax.experimental import pallas as pl
from jax.experimental.pallas import tpu as pltpu
from jax.experimental.pallas import tpu_sc as plsc

EMBED = 64
SUB = 16
NH, ND, NM, NO = 24, 7, 12, 2
NROWS = NH * ND * NM * NO
TOTAL = 4096 * 200
NC, NS = 2, 16
NW = NC * NS
PER_W = TOTAL // NW
CHUNK = 6400
NCHUNK = PER_W // CHUNK
BATCH = 128
NB = PER_W // BATCH
NSLOT = 8
LOOK = 4


def _fuse_body(ht_ref, dt_ref, mt_ref, ot_ref, w_ref, b_ref, out_ref):
    i = lax.broadcasted_iota(jnp.int32, (NROWS, 1), 0)
    parts = [
        (i // (ND * NM * NO), ht_ref, NH, 0),
        ((i // (NM * NO)) % ND, dt_ref, ND, 1),
        ((i // NO) % NM, mt_ref, NM, 2),
        (i % NO, ot_ref, NO, 3),
    ]
    acc = jnp.broadcast_to(b_ref[...], (NROWS, EMBED))
    for idx, tab_ref, n, j in parts:
        w_slice = w_ref[:, j * SUB:(j + 1) * SUB]
        pt = lax.dot_general(tab_ref[...], w_slice, (((1,), (1,)), ((), ())),
                             preferred_element_type=jnp.float32,
                             precision=lax.Precision.HIGHEST)
        oh = (idx == lax.broadcasted_iota(jnp.int32, (NROWS, n), 1)).astype(jnp.float32)
        acc = acc + jnp.dot(oh, pt, preferred_element_type=jnp.float32,
                            precision=lax.Precision.HIGHEST)
    out_ref[...] = acc


def _fuse_tables(ht, dt, mt, ot, W, b):
    return pl.pallas_call(
        _fuse_body,
        out_shape=jax.ShapeDtypeStruct((NROWS, EMBED), jnp.float32),
    )(ht, dt, mt, ot, W, b.reshape(1, EMBED))


@functools.cache
def _make_sc_gather():
    mesh = plsc.VectorSubcoreMesh(core_axis_name="c", subcore_axis_name="s")

    @functools.partial(
        pl.kernel,
        mesh=mesh,
        compiler_params=pltpu.CompilerParams(use_tc_tiling_on_sc=False),
        out_type=jax.ShapeDtypeStruct((TOTAL, EMBED), jnp.float32),
        scratch_types=[
            pltpu.VMEM((CHUNK,), jnp.int32),
            pltpu.VMEM((CHUNK,), jnp.int32),
            pltpu.VMEM((CHUNK,), jnp.int32),
            pltpu.VMEM((CHUNK,), jnp.int32),
            pltpu.VMEM((PER_W,), jnp.int32),
            pltpu.VMEM((NSLOT, BATCH, EMBED), jnp.float32),
            pltpu.SemaphoreType.DMA,
            pltpu.SemaphoreType.DMA((NSLOT,)),
            pltpu.SemaphoreType.DMA((NSLOT,)),
        ],
    )
    def _sc_gather(h_hbm, d_hbm, m_hbm, o_hbm, tab_hbm, out_hbm,
                   h_v, d_v, m_v, o_v, f_v, rows_v, isem, gsem, ssem):
        wid = lax.axis_index("s") * NC + lax.axis_index("c")
        base = wid * PER_W

        for c in range(NCHUNK):
            start = base + c * CHUNK
            cps = [pltpu.async_copy(src.at[pl.ds(start, CHUNK)], dst, isem)
                   for src, dst in ((h_hbm, h_v), (d_hbm, d_v),
                                    (m_hbm, m_v), (o_hbm, o_v))]
            for cp in cps:
                cp.wait()

            def vec_body(vi, _, c=c):
                sl = pl.ds(vi * 16, 16)
                f_v[pl.ds(c * CHUNK + vi * 16, 16)] = (
                    ((h_v[sl] * ND + d_v[sl]) * NM + m_v[sl]) * NO + o_v[sl])
                return 0

            lax.fori_loop(0, CHUNK // 16, vec_body, 0)

        def gather_desc(r, s):
            idx = f_v.at[pl.ds(r * BATCH, BATCH)]
            return pltpu.make_async_copy(tab_hbm.at[idx], rows_v.at[s],
                                         gsem.at[s])

        def scatter_desc(r, s):
            dst = out_hbm.at[pl.ds(base + r * BATCH, BATCH)]
            return pltpu.make_async_copy(rows_v.at[s], dst, ssem.at[s])

        for r in range(LOOK):
            gather_desc(r, r).start()

        def round_group(g, _):
            for s in range(NSLOT):
                bi = g * NSLOT + s
                r = bi + LOOK
                sl = (s + LOOK) % NSLOT

                @pl.when(jnp.logical_and(r < NB, r >= NSLOT))
                def _():
                    scatter_desc(r - NSLOT, sl).wait()

                @pl.when(r < NB)
                def _():
                    gather_desc(r, sl).start()

                gather_desc(bi, s).wait()
                scatter_desc(bi, s).start()
            return 0

        lax.fori_loop(0, NB // NSLOT, round_group, 0)

        for s in range(NSLOT):
            scatter_desc(NB - NSLOT + s, s).wait()

    return _sc_gather


def kernel(hours, days, months, holidays, hour_table, day_table, month_table,
           holiday_table, W, b):
    fused = _fuse_tables(hour_table, day_table, month_table, holiday_table, W, b)
    h = hours.reshape(TOTAL).astype(jnp.int32)
    d = days.reshape(TOTAL).astype(jnp.int32)
    m = months.reshape(TOTAL).astype(jnp.int32)
    o = holidays.reshape(TOTAL).astype(jnp.int32)
    out = _make_sc_gather()(h, d, m, o, fused)
    return out.reshape(hours.shape[0], hours.shape[1], EMBED)


# speedup vs baseline: 22.6850x; 1.5475x over previous
import functools

import jax
import jax.numpy as jnp
from jax import lax
from jax.experimental import pallas as pl
from jax.experimental.pallas import tpu as pltpu
from jax.experimental.pallas import tpu_sc as plsc

EMBED = 64
SUB = 16
NH, ND, NM, NO = 24, 7, 12, 2
NROWS = NH * ND * NM * NO
TPAD = 4096
B, S = 4096, 200
NC, NS = 2, 16
NW = NC * NS
NEG = 4
NBP = NW // NEG
SB_TILES = 25
TILES = 4 * SB_TILES


def _fuse_body(ht_ref, dt_ref, mt_ref, ot_ref, w_ref, b_ref, out_ref):
    i = lax.broadcasted_iota(jnp.int32, (NROWS, 1), 0)
    parts = [
        (i // (ND * NM * NO), ht_ref, NH, 0),
        ((i // (NM * NO)) % ND, dt_ref, ND, 1),
        ((i // NO) % NM, mt_ref, NM, 2),
        (i % NO, ot_ref, NO, 3),
    ]
    acc = jnp.broadcast_to(b_ref[...], (EMBED, NROWS))
    for idx, tab_ref, n, j in parts:
        w_slice = w_ref[:, j * SUB:(j + 1) * SUB]
        pt = lax.dot_general(tab_ref[...], w_slice, (((1,), (1,)), ((), ())),
                             preferred_element_type=jnp.float32,
                             precision=lax.Precision.HIGHEST)
        oh = (idx == lax.broadcasted_iota(jnp.int32, (NROWS, n), 1)).astype(jnp.float32)
        acc = acc + lax.dot_general(pt, oh, (((0,), (1,)), ((), ())),
                                    preferred_element_type=jnp.float32,
                                    precision=lax.Precision.HIGHEST)
    out_ref[...] = acc


def _fuse_tables(ht, dt, mt, ot, W, b):
    return pl.pallas_call(
        _fuse_body,
        out_shape=jax.ShapeDtypeStruct((EMBED, NROWS), jnp.float32),
    )(ht, dt, mt, ot, W, b.reshape(EMBED, 1))


@functools.cache
def _make_sc_gather():
    mesh = plsc.VectorSubcoreMesh(core_axis_name="c", subcore_axis_name="s")

    @functools.partial(
        pl.kernel,
        mesh=mesh,
        compiler_params=pltpu.CompilerParams(use_tc_tiling_on_sc=False,
                                             needs_layout_passes=False),
        out_type=jax.ShapeDtypeStruct((S, EMBED // 8, B // 128, 8, 128),
                                      jnp.float32),
        scratch_types=[
            pltpu.VMEM((16 * TPAD,), jnp.float32),
            pltpu.VMEM((2, 4, 8, 128), jnp.int32),
            pltpu.VMEM((2, 8, 16, 128), jnp.float32),
            pltpu.SemaphoreType.DMA,
            pltpu.SemaphoreType.DMA,
            pltpu.SemaphoreType.DMA,
            pltpu.SemaphoreType.DMA,
            pltpu.SemaphoreType.DMA,
        ],
    )
    def _sc_gather(h4, d4, m4, o4, tab_hbm, out5,
                   tabv, ibuf, obuf, tsem, isem0, isem1, ssem0, ssem1):
        wid = lax.axis_index("s") * NC + lax.axis_index("c")
        eg = wid % NEG
        bp = wid // NEG
        isems = (isem0, isem1)
        ssems = (ssem0, ssem1)

        tcps = [pltpu.make_async_copy(tab_hbm.at[eg * 16 + e],
                                      tabv.at[pl.ds(e * TPAD, NROWS)], tsem)
                for e in range(16)]
        for cp in tcps:
            cp.start()
        for cp in tcps:
            cp.wait()

        def tile_coords(t):
            return bp * 4 + t // SB_TILES, t % SB_TILES

        def idx_copies(t, slot):
            b0, s0 = tile_coords(t)
            sem = isems[slot]
            return [pltpu.make_async_copy(src.at[s0, b0], ibuf.at[slot, j], sem)
                    for j, src in enumerate((h4, d4, m4, o4))]

        def scatter_copies(t, slot):
            b0, s0 = tile_coords(t)
            sem = ssems[slot]
            return [pltpu.make_async_copy(
                obuf.at[slot, :, pl.ds(h * 8, 8), :],
                out5.at[pl.ds(s0 * 8, 8), eg * 2 + h, b0],
                sem) for h in range(2)]

        for cp in idx_copies(0, 0):
            cp.start()

        ebase = [jnp.full((16,), e * TPAD, jnp.int32) for e in range(16)]

        def group_body(g, _):
            for p in range(2):
                t = g * 2 + p

                for cp in idx_copies(t, p):
                    cp.wait()

                @pl.when(t + 1 < TILES)
                def _():
                    for cp in idx_copies(t + 1, 1 - p):
                        cp.start()

                @pl.when(t >= 2)
                def _():
                    for cp in scatter_copies(t - 2, p):
                        cp.wait()

                def sr_body(sr, _):
                    fv = []
                    for brg in range(8):
                        sl = pl.ds(brg * 16, 16)
                        hv = ibuf[p, 0, sr, sl]
                        dv = ibuf[p, 1, sr, sl]
                        mv = ibuf[p, 2, sr, sl]
                        ov = ibuf[p, 3, sr, sl]
                        fv.append(((hv * ND + dv) * NM + mv) * NO + ov)
                    for e in range(16):
                        for brg in range(8):
                            val = plsc.load_gather(tabv, [fv[brg] + ebase[e]])
                            obuf[p, sr, e, pl.ds(brg * 16, 16)] = val
                    return 0

                lax.fori_loop(0, 8, sr_body, 0)

                for cp in scatter_copies(t, p):
                    cp.start()
            return 0

        lax.fori_loop(0, TILES // 2, group_body, 0)

        for t in (TILES - 2, TILES - 1):
            for cp in scatter_copies(t, t % 2):
                cp.wait()

    return _sc_gather


def _as_tiles(x):
    return x.astype(jnp.int32).T.reshape(SB_TILES, 8, B // 128, 128).transpose(0, 2, 1, 3)


def kernel(hours, days, months, holidays, hour_table, day_table, month_table,
           holiday_table, W, b):
    fused_t = _fuse_tables(hour_table, day_table, month_table, holiday_table, W, b)
    out5 = _make_sc_gather()(_as_tiles(hours), _as_tiles(days),
                             _as_tiles(months), _as_tiles(holidays), fused_t)
    return out5.transpose(2, 4, 0, 1, 3).reshape(B, S, EMBED)


# speedup vs baseline: 27.2324x; 1.2005x over previous
import functools

import jax
import jax.numpy as jnp
from jax import lax
from jax.experimental import pallas as pl
from jax.experimental.pallas import tpu as pltpu
from jax.experimental.pallas import tpu_sc as plsc

EMBED = 64
SUB = 16
NH, ND, NM, NO = 24, 7, 12, 2
NROWS = NH * ND * NM * NO
TPAD = 4096
B, S = 4096, 200
NC, NS = 2, 16
NW = NC * NS
NEG = 4
NBP = NW // NEG
SB_TILES = 25
TILES = 4 * SB_TILES


def _fuse_body(ht_ref, dt_ref, mt_ref, ot_ref, w_ref, b_ref, out_ref):
    i = lax.broadcasted_iota(jnp.int32, (NROWS, 1), 0)
    parts = [
        (i // (ND * NM * NO), ht_ref, NH, 0),
        ((i // (NM * NO)) % ND, dt_ref, ND, 1),
        ((i // NO) % NM, mt_ref, NM, 2),
        (i % NO, ot_ref, NO, 3),
    ]
    acc = jnp.broadcast_to(b_ref[...], (EMBED, NROWS))
    for idx, tab_ref, n, j in parts:
        w_slice = w_ref[:, j * SUB:(j + 1) * SUB]
        pt = lax.dot_general(tab_ref[...], w_slice, (((1,), (1,)), ((), ())),
                             preferred_element_type=jnp.float32,
                             precision=lax.Precision.HIGHEST)
        oh = (idx == lax.broadcasted_iota(jnp.int32, (NROWS, n), 1)).astype(jnp.float32)
        acc = acc + lax.dot_general(pt, oh, (((0,), (1,)), ((), ())),
                                    preferred_element_type=jnp.float32,
                                    precision=lax.Precision.HIGHEST)
    out_ref[...] = acc


def _fuse_tables(ht, dt, mt, ot, W, b):
    return pl.pallas_call(
        _fuse_body,
        out_shape=jax.ShapeDtypeStruct((EMBED, NROWS), jnp.float32),
    )(ht, dt, mt, ot, W, b.reshape(EMBED, 1))


@functools.cache
def _make_sc_gather():
    mesh = plsc.VectorSubcoreMesh(core_axis_name="c", subcore_axis_name="s")

    @functools.partial(
        pl.kernel,
        mesh=mesh,
        compiler_params=pltpu.CompilerParams(use_tc_tiling_on_sc=False,
                                             needs_layout_passes=False),
        out_type=jax.ShapeDtypeStruct((S, EMBED // 8, B // 128, 8, 128),
                                      jnp.float32),
        scratch_types=[
            pltpu.VMEM((16 * TPAD,), jnp.float32),
            pltpu.VMEM((2, 4, 8, 128), jnp.int32),
            pltpu.VMEM((2, 8, 16, 128), jnp.float32),
            pltpu.SemaphoreType.DMA,
            pltpu.SemaphoreType.DMA,
            pltpu.SemaphoreType.DMA,
            pltpu.SemaphoreType.DMA,
            pltpu.SemaphoreType.DMA,
        ],
    )
    def _sc_gather(h4, d4, m4, o4, tab_hbm, out5,
                   tabv, ibuf, obuf, tsem, isem0, isem1, ssem0, ssem1):
        wid = lax.axis_index("s") * NC + lax.axis_index("c")
        eg = wid % NEG
        bp = wid // NEG
        isems = (isem0, isem1)
        ssems = (ssem0, ssem1)

        tcps = [pltpu.make_async_copy(tab_hbm.at[eg * 16 + e],
                                      tabv.at[pl.ds(e * TPAD, NROWS)], tsem)
                for e in range(16)]
        for cp in tcps:
            cp.start()
        for cp in tcps:
            cp.wait()

        def tile_coords(t):
            return bp * 4 + t // SB_TILES, t % SB_TILES

        def idx_copies(t, slot):
            b0, s0 = tile_coords(t)
            sem = isems[slot]
            return [pltpu.make_async_copy(src.at[s0, b0], ibuf.at[slot, j], sem)
                    for j, src in enumerate((h4, d4, m4, o4))]

        def scatter_copies(t, slot):
            b0, s0 = tile_coords(t)
            sem = ssems[slot]
            return [pltpu.make_async_copy(
                obuf.at[slot, :, pl.ds(h * 8, 8), :],
                out5.at[pl.ds(s0 * 8, 8), eg * 2 + h, b0],
                sem) for h in range(2)]

        for cp in idx_copies(0, 0):
            cp.start()

        ebase = [jnp.full((16,), e * TPAD, jnp.int32) for e in range(16)]

        def group_body(g, _):
            for p in range(2):
                t = g * 2 + p

                for cp in idx_copies(t, p):
                    cp.wait()

                @pl.when(t + 1 < TILES)
                def _():
                    for cp in idx_copies(t + 1, 1 - p):
                        cp.start()

                @pl.when(t >= 2)
                def _():
                    for cp in scatter_copies(t - 2, p):
                        cp.wait()

                def sr_body(sr, _):
                    fv = []
                    for brg in range(8):
                        sl = pl.ds(brg * 16, 16)
                        hv = ibuf[p, 0, sr, sl]
                        dv = ibuf[p, 1, sr, sl]
                        mv = ibuf[p, 2, sr, sl]
                        ov = ibuf[p, 3, sr, sl]
                        fv.append(((hv * ND + dv) * NM + mv) * NO + ov)
                    iotav = lax.iota(jnp.int32, 16)
                    for e in range(16):
                        for brg in range(8):
                            idx = jnp.minimum(fv[brg], 0) + iotav + ebase[e]
                            val = plsc.load_gather(tabv, [idx])
                            obuf[p, sr, e, pl.ds(brg * 16, 16)] = val
                    return 0

                lax.fori_loop(0, 8, sr_body, 0)

                for cp in scatter_copies(t, p):
                    cp.start()
            return 0

        lax.fori_loop(0, TILES // 2, group_body, 0)

        for t in (TILES - 2, TILES - 1):
            for cp in scatter_copies(t, t % 2):
                cp.wait()

    return _sc_gather


def _as_tiles(x):
    return x.astype(jnp.int32).T.reshape(SB_TILES, 8, B // 128, 128).transpose(0, 2, 1, 3)


def kernel(hours, days, months, holidays, hour_table, day_table, month_table,
           holiday_table, W, b):
    fused_t = _fuse_tables(hour_table, day_table, month_table, holiday_table, W, b)
    out5 = _make_sc_gather()(_as_tiles(hours), _as_tiles(days),
                             _as_tiles(months), _as_tiles(holidays), fused_t)
    return out5.transpose(2, 4, 0, 1, 3).reshape(B, S, EMBED)


# speedup vs baseline: 34.0691x; 1.2511x over previous
import functools

import jax
import jax.numpy as jnp
from jax import lax
from jax.experimental import pallas as pl
from jax.experimental.pallas import tpu as pltpu
from jax.experimental.pallas import tpu_sc as plsc

EMBED = 64
SUB = 16
NH, ND, NM, NO = 24, 7, 12, 2
NROWS = NH * ND * NM * NO
TPAD = 4096
B, S = 4096, 200
NC, NS = 2, 16
NW = NC * NS
NEG = 4
NBP = NW // NEG
SB_TILES = 25
TILES = 4 * SB_TILES


def _fuse_body(ht_ref, dt_ref, mt_ref, ot_ref, w_ref, b_ref, out_ref):
    i = lax.broadcasted_iota(jnp.int32, (NROWS, 1), 0)
    parts = [
        (i // (ND * NM * NO), ht_ref, NH, 0),
        ((i // (NM * NO)) % ND, dt_ref, ND, 1),
        ((i // NO) % NM, mt_ref, NM, 2),
        (i % NO, ot_ref, NO, 3),
    ]
    acc = jnp.broadcast_to(b_ref[...], (EMBED, NROWS))
    for idx, tab_ref, n, j in parts:
        w_slice = w_ref[:, j * SUB:(j + 1) * SUB]
        pt = lax.dot_general(tab_ref[...], w_slice, (((1,), (1,)), ((), ())),
                             preferred_element_type=jnp.float32,
                             precision=lax.Precision.HIGHEST)
        oh = (idx == lax.broadcasted_iota(jnp.int32, (NROWS, n), 1)).astype(jnp.float32)
        acc = acc + lax.dot_general(pt, oh, (((0,), (1,)), ((), ())),
                                    preferred_element_type=jnp.float32,
                                    precision=lax.Precision.HIGHEST)
    out_ref[...] = acc


def _fuse_tables(ht, dt, mt, ot, W, b):
    return pl.pallas_call(
        _fuse_body,
        out_shape=jax.ShapeDtypeStruct((EMBED, NROWS), jnp.float32),
    )(ht, dt, mt, ot, W, b.reshape(EMBED, 1))


@functools.cache
def _make_sc_gather():
    mesh = plsc.VectorSubcoreMesh(core_axis_name="c", subcore_axis_name="s")

    @functools.partial(
        pl.kernel,
        mesh=mesh,
        compiler_params=pltpu.CompilerParams(use_tc_tiling_on_sc=False,
                                             needs_layout_passes=False),
        out_type=jax.ShapeDtypeStruct((S, EMBED // 8, B // 128, 8, 128),
                                      jnp.float32),
        scratch_types=[
            pltpu.VMEM((16 * TPAD,), jnp.float32),
            pltpu.VMEM((2, 4, 8, 128), jnp.int32),
            pltpu.VMEM((2, 8, 16, 128), jnp.float32),
            pltpu.SemaphoreType.DMA,
            pltpu.SemaphoreType.DMA,
            pltpu.SemaphoreType.DMA,
            pltpu.SemaphoreType.DMA,
            pltpu.SemaphoreType.DMA,
        ],
    )
    def _sc_gather(h4, d4, m4, o4, tab_hbm, out5,
                   tabv, ibuf, obuf, tsem, isem0, isem1, ssem0, ssem1):
        wid = lax.axis_index("s") * NC + lax.axis_index("c")
        eg = wid % NEG
        bp = wid // NEG
        isems = (isem0, isem1)
        ssems = (ssem0, ssem1)

        tcps = [pltpu.make_async_copy(tab_hbm.at[eg * 16 + e],
                                      tabv.at[pl.ds(e * TPAD, NROWS)], tsem)
                for e in range(16)]
        for cp in tcps:
            cp.start()
        for cp in tcps:
            cp.wait()

        def tile_coords(t):
            return bp * 4 + t // SB_TILES, t % SB_TILES

        def idx_copies(t, slot):
            b0, s0 = tile_coords(t)
            sem = isems[slot]
            return [pltpu.make_async_copy(src.at[s0, b0], ibuf.at[slot, j], sem)
                    for j, src in enumerate((h4, d4, m4, o4))]

        def scatter_copies(t, slot):
            b0, s0 = tile_coords(t)
            sem = ssems[slot]
            return [pltpu.make_async_copy(
                obuf.at[slot, :, pl.ds(h * 8, 8), :],
                out5.at[pl.ds(s0 * 8, 8), eg * 2 + h, b0],
                sem) for h in range(2)]

        for cp in idx_copies(0, 0):
            cp.start()

        ebase = [jnp.full((16,), e * TPAD, jnp.int32) for e in range(16)]

        def group_body(g, _):
            for p in range(2):
                t = g * 2 + p

                for cp in idx_copies(t, p):
                    cp.wait()

                @pl.when(t + 1 < TILES)
                def _():
                    for cp in idx_copies(t + 1, 1 - p):
                        cp.start()

                @pl.when(t >= 2)
                def _():
                    for cp in scatter_copies(t - 2, p):
                        cp.wait()

                @plsc.parallel_loop(0, 8, 1, unroll=2)
                def sr_body(sr):
                    fv = []
                    for brg in range(8):
                        sl = pl.ds(brg * 16, 16)
                        hv = ibuf[p, 0, sr, sl]
                        dv = ibuf[p, 1, sr, sl]
                        mv = ibuf[p, 2, sr, sl]
                        ov = ibuf[p, 3, sr, sl]
                        fv.append((hv * (ND * NM * NO) + dv * (NM * NO))
                                  + (mv * NO + ov))
                    for e in range(16):
                        for brg in range(8):
                            val = plsc.load_gather(tabv, [fv[brg] + ebase[e]])
                            obuf[p, sr, e, pl.ds(brg * 16, 16)] = val

                for cp in scatter_copies(t, p):
                    cp.start()
            return 0

        lax.fori_loop(0, TILES // 2, group_body, 0)

        for t in (TILES - 2, TILES - 1):
            for cp in scatter_copies(t, t % 2):
                cp.wait()

    return _sc_gather


def _as_tiles(x):
    return x.astype(jnp.int32).T.reshape(SB_TILES, 8, B // 128, 128).transpose(0, 2, 1, 3)


def kernel(hours, days, months, holidays, hour_table, day_table, month_table,
           holiday_table, W, b):
    fused_t = _fuse_tables(hour_table, day_table, month_table, holiday_table, W, b)
    out5 = _make_sc_gather()(_as_tiles(hours), _as_tiles(days),
                             _as_tiles(months), _as_tiles(holidays), fused_t)
    return out5.transpose(2, 4, 0, 1, 3).reshape(B, S, EMBED)
